# baseline (speedup 1.0000x reference)
import functools

import jax
import jax.numpy as jnp
from jax import lax
from jax.experimental import pallas as pl
from jax.experimental.pallas import tpu as pltpu
from jax.experimental.pallas import tpu_sc as plsc

B, L, DIM = 16384, 20, 128
ZCH_0, ZCH_1 = 8, 16
NBINS = ZCH_0 + ZCH_1

NC, NS, LANES = 2, 16, 16
NW = NC * NS

NCHUNK = 2
HB = B // NCHUNK
BPT = HB // NW
GROUPS = BPT // LANES

BLK = 2048


def _hist_body(chunk, ids0_hbm, ids1_hbm, cnt_hbm, ids0_v, ids1_v, cnt_v):
    wid = lax.axis_index("s") * NC + lax.axis_index("c")
    base = chunk * HB + wid * BPT

    pltpu.sync_copy(ids0_hbm.at[:, pl.ds(base, BPT)], ids0_v)
    pltpu.sync_copy(ids1_hbm.at[:, pl.ds(base, BPT)], ids1_v)

    iota = lax.iota(jnp.int32, LANES)
    zeros = jnp.zeros((LANES,), jnp.float32)
    ones = zeros + 1.0

    def zcol(i, _):
        for r in range(NBINS):
            cnt_v[r, pl.ds(i * LANES, LANES)] = zeros
        return 0

    lax.fori_loop(0, BPT // LANES, zcol, 0)

    def group(g, _):
        lbag = g * LANES + iota
        col = g * LANES
        for l in range(L):
            g0 = ids0_v[l, pl.ds(col, LANES)]
            g1 = ids1_v[l, pl.ds(col, LANES)]
            e0 = g0 & (ZCH_0 - 1)
            e1 = (g1 & (ZCH_1 - 1)) + ZCH_0
            plsc.addupdate_scatter(cnt_v, [e0, lbag], ones)
            plsc.addupdate_scatter(cnt_v, [e1, lbag], ones)
        return 0

    lax.fori_loop(0, GROUPS, group, 0)

    pltpu.sync_copy(cnt_v, cnt_hbm.at[:, pl.ds(wid * BPT, BPT)])


@functools.lru_cache(maxsize=None)
def _hist(chunk):
    return pl.kernel(
        functools.partial(_hist_body, chunk),
        out_type=jax.ShapeDtypeStruct((NBINS, HB), jnp.float32),
        mesh=plsc.VectorSubcoreMesh(
            core_axis_name="c", subcore_axis_name="s", num_cores=NC, num_subcores=NS
        ),
        compiler_params=pltpu.CompilerParams(needs_layout_passes=False),
        scratch_types=[
            pltpu.VMEM((L, BPT), jnp.int32),
            pltpu.VMEM((L, BPT), jnp.int32),
            pltpu.VMEM((NBINS, BPT), jnp.float32),
        ],
    )


def _mm_block(ct_ref, t0_ref, t1_ref, out_ref, loss_ref, i):
    ct = ct_ref[...]
    zz = jnp.zeros((ZCH_0, DIM), jnp.float32)
    w = jnp.concatenate(
        [
            jnp.concatenate([t0_ref[...], zz], axis=1),
            jnp.concatenate([jnp.zeros((ZCH_1, DIM), jnp.float32), t1_ref[...]], axis=1),
        ],
        axis=0,
    )
    ct_bf = ct.astype(jnp.bfloat16)
    w_hi = w.astype(jnp.bfloat16)
    w_lo = (w - w_hi.astype(jnp.float32)).astype(jnp.bfloat16)
    dn = (((0,), (0,)), ((), ()))
    p = lax.dot_general(ct_bf, w_hi, dn, preferred_element_type=jnp.float32)
    p = p + lax.dot_general(ct_bf, w_lo, dn, preferred_element_type=jnp.float32)
    out_ref[...] = p

    @pl.when(i == 0)
    def _():
        loss_ref[0, 0] = 0.0

    loss_ref[0, 0] += jnp.sum(p)


def _mm_body_a(ct_ref, t0_ref, t1_ref, out_ref, loss_ref):
    _mm_block(ct_ref, t0_ref, t1_ref, out_ref, loss_ref, pl.program_id(0))


def _mm_body_b(ct_ref, t0_ref, t1_ref, pred_in_ref, out_ref, loss_ref):
    del pred_in_ref
    _mm_block(ct_ref, t0_ref, t1_ref, out_ref, loss_ref, pl.program_id(0))


def _pool_matmul_a(ct, table_0, table_1):
    return pl.pallas_call(
        _mm_body_a,
        grid=(HB // BLK,),
        in_specs=[
            pl.BlockSpec((NBINS, BLK), lambda i: (0, i)),
            pl.BlockSpec((ZCH_0, DIM), lambda i: (0, 0)),
            pl.BlockSpec((ZCH_1, DIM), lambda i: (0, 0)),
        ],
        out_specs=[
            pl.BlockSpec((BLK, 2 * DIM), lambda i: (i, 0)),
            pl.BlockSpec((1, 1), lambda i: (0, 0), memory_space=pltpu.SMEM),
        ],
        out_shape=[
            jax.ShapeDtypeStruct((B, 2 * DIM), jnp.float32),
            jax.ShapeDtypeStruct((1, 1), jnp.float32),
        ],
    )(ct, table_0, table_1)


def _pool_matmul_b(ct, table_0, table_1, pred):
    return pl.pallas_call(
        _mm_body_b,
        grid=(HB // BLK,),
        in_specs=[
            pl.BlockSpec((NBINS, BLK), lambda i: (0, i)),
            pl.BlockSpec((ZCH_0, DIM), lambda i: (0, 0)),
            pl.BlockSpec((ZCH_1, DIM), lambda i: (0, 0)),
            pl.BlockSpec(memory_space=pl.ANY),
        ],
        out_specs=[
            pl.BlockSpec((BLK, 2 * DIM), lambda i: (i + HB // BLK, 0)),
            pl.BlockSpec((1, 1), lambda i: (0, 0), memory_space=pltpu.SMEM),
        ],
        out_shape=[
            jax.ShapeDtypeStruct((B, 2 * DIM), jnp.float32),
            jax.ShapeDtypeStruct((1, 1), jnp.float32),
        ],
        input_output_aliases={3: 0},
    )(ct, table_0, table_1, pred)


@jax.jit
def kernel(ids_0, ids_1, table_0, table_1):
    ids0_t = ids_0.astype(jnp.int32).T
    ids1_t = ids_1.astype(jnp.int32).T
    ct0 = _hist(0)(ids0_t, ids1_t)
    ct1 = _hist(1)(ids0_t, ids1_t)
    pred, loss_a = _pool_matmul_a(ct0, table_0, table_1)
    pred, loss_b = _pool_matmul_b(ct1, table_0, table_1, pred)
    loss = (loss_a[0, 0] + loss_b[0, 0]) / (B * 2 * DIM)
    return (loss, pred)


# speedup vs baseline: 1.1160x; 1.1160x over previous
import functools

import jax
import jax.numpy as jnp
from jax import lax
from jax.experimental import pallas as pl
from jax.experimental.pallas import tpu as pltpu
from jax.experimental.pallas import tpu_sc as plsc

B, L, DIM = 16384, 20, 128
ZCH_0, ZCH_1 = 8, 16
NBINS = ZCH_0 + ZCH_1

NC, NS, LANES = 2, 16, 16
NW = NC * NS
BPT = B // NW
GROUPS = BPT // LANES

BLK = 4096


def _hist_body(ids0_hbm, ids1_hbm, cnt_hbm, ids0_v, ids1_v, cnt_v):
    wid = lax.axis_index("s") * NC + lax.axis_index("c")
    base = wid * BPT

    pltpu.sync_copy(ids0_hbm.at[:, pl.ds(base, BPT)], ids0_v)
    pltpu.sync_copy(ids1_hbm.at[:, pl.ds(base, BPT)], ids1_v)

    iota = lax.iota(jnp.int32, LANES)
    zeros = jnp.zeros((LANES,), jnp.float32)
    ones = zeros + 1.0

    def zcol(i, _):
        for r in range(NBINS):
            cnt_v[r, pl.ds(i * LANES, LANES)] = zeros
        return 0

    lax.fori_loop(0, BPT // LANES, zcol, 0)

    def group(g, _):
        lbag = g * LANES + iota
        col = g * LANES
        for l in range(L):
            g0 = ids0_v[l, pl.ds(col, LANES)]
            g1 = ids1_v[l, pl.ds(col, LANES)]
            e0 = g0 & (ZCH_0 - 1)
            e1 = (g1 & (ZCH_1 - 1)) + ZCH_0
            plsc.addupdate_scatter(cnt_v, [e0, lbag], ones)
            plsc.addupdate_scatter(cnt_v, [e1, lbag], ones)
        return 0

    lax.fori_loop(0, GROUPS, group, 0)

    pltpu.sync_copy(cnt_v, cnt_hbm.at[:, pl.ds(base, BPT)])


@functools.lru_cache(maxsize=None)
def _hist():
    return pl.kernel(
        _hist_body,
        out_type=jax.ShapeDtypeStruct((NBINS, B), jnp.float32),
        mesh=plsc.VectorSubcoreMesh(
            core_axis_name="c", subcore_axis_name="s", num_cores=NC, num_subcores=NS
        ),
        compiler_params=pltpu.CompilerParams(needs_layout_passes=False),
        scratch_types=[
            pltpu.VMEM((L, BPT), jnp.int32),
            pltpu.VMEM((L, BPT), jnp.int32),
            pltpu.VMEM((NBINS, BPT), jnp.float32),
        ],
    )


def _mm_body(ct_ref, t0_ref, t1_ref, out_ref, loss_ref):
    i = pl.program_id(0)
    ct = ct_ref[...]
    zz = jnp.zeros((ZCH_0, DIM), jnp.float32)
    w = jnp.concatenate(
        [
            jnp.concatenate([t0_ref[...], zz], axis=1),
            jnp.concatenate([jnp.zeros((ZCH_1, DIM), jnp.float32), t1_ref[...]], axis=1),
        ],
        axis=0,
    )
    ct_bf = ct.astype(jnp.bfloat16)
    w_hi = w.astype(jnp.bfloat16)
    w_lo = (w - w_hi.astype(jnp.float32)).astype(jnp.bfloat16)
    dn = (((0,), (0,)), ((), ()))
    p = lax.dot_general(ct_bf, w_hi, dn, preferred_element_type=jnp.float32)
    p = p + lax.dot_general(ct_bf, w_lo, dn, preferred_element_type=jnp.float32)
    out_ref[...] = p

    @pl.when(i == 0)
    def _():
        loss_ref[0, 0] = 0.0

    csum = jnp.sum(ct, axis=1, keepdims=True)
    wsum = jnp.sum(w, axis=1, keepdims=True)
    loss_ref[0, 0] += jnp.sum(csum * wsum)


def _pool_matmul(ct, table_0, table_1):
    return pl.pallas_call(
        _mm_body,
        grid=(B // BLK,),
        in_specs=[
            pl.BlockSpec((NBINS, BLK), lambda i: (0, i)),
            pl.BlockSpec((ZCH_0, DIM), lambda i: (0, 0)),
            pl.BlockSpec((ZCH_1, DIM), lambda i: (0, 0)),
        ],
        out_specs=[
            pl.BlockSpec((BLK, 2 * DIM), lambda i: (i, 0)),
            pl.BlockSpec((1, 1), lambda i: (0, 0), memory_space=pltpu.SMEM),
        ],
        out_shape=[
            jax.ShapeDtypeStruct((B, 2 * DIM), jnp.float32),
            jax.ShapeDtypeStruct((1, 1), jnp.float32),
        ],
    )(ct, table_0, table_1)


@jax.jit
def kernel(ids_0, ids_1, table_0, table_1):
    ids0_t = ids_0.astype(jnp.int32).T
    ids1_t = ids_1.astype(jnp.int32).T
    ct = _hist()(ids0_t, ids1_t)
    pred, loss_sum = _pool_matmul(ct, table_0, table_1)
    loss = loss_sum[0, 0] / (B * 2 * DIM)
    return (loss, pred)


# speedup vs baseline: 1.1175x; 1.0013x over previous
import functools

import jax
import jax.numpy as jnp
from jax import lax
from jax.experimental import pallas as pl
from jax.experimental.pallas import tpu as pltpu
from jax.experimental.pallas import tpu_sc as plsc

B, L, DIM = 16384, 20, 128
ZCH_0, ZCH_1 = 8, 16
NBINS = ZCH_0 + ZCH_1

NC, NS, LANES = 2, 16, 16
NW = NC * NS
BPT = B // NW
GROUPS = BPT // LANES

BLK = 8192


def _hist_body(ids0_hbm, ids1_hbm, cnt_hbm, ids0_v, ids1_v, cnt_v):
    wid = lax.axis_index("s") * NC + lax.axis_index("c")
    base = wid * BPT

    pltpu.sync_copy(ids0_hbm.at[:, pl.ds(base, BPT)], ids0_v)
    pltpu.sync_copy(ids1_hbm.at[:, pl.ds(base, BPT)], ids1_v)

    iota = lax.iota(jnp.int32, LANES)
    zeros = jnp.zeros((LANES,), jnp.float32)
    ones = zeros + 1.0

    def zcol(i, _):
        for r in range(NBINS):
            cnt_v[r, pl.ds(i * LANES, LANES)] = zeros
        return 0

    lax.fori_loop(0, BPT // LANES, zcol, 0)

    def group(g, _):
        lbag = g * LANES + iota
        col = g * LANES
        for l in range(L):
            g0 = ids0_v[l, pl.ds(col, LANES)]
            g1 = ids1_v[l, pl.ds(col, LANES)]
            e0 = g0 & (ZCH_0 - 1)
            e1 = (g1 & (ZCH_1 - 1)) + ZCH_0
            plsc.addupdate_scatter(cnt_v, [e0, lbag], ones)
            plsc.addupdate_scatter(cnt_v, [e1, lbag], ones)
        return 0

    lax.fori_loop(0, GROUPS, group, 0)

    pltpu.sync_copy(cnt_v, cnt_hbm.at[:, pl.ds(base, BPT)])


@functools.lru_cache(maxsize=None)
def _hist():
    return pl.kernel(
        _hist_body,
        out_type=jax.ShapeDtypeStruct((NBINS, B), jnp.float32),
        mesh=plsc.VectorSubcoreMesh(
            core_axis_name="c", subcore_axis_name="s", num_cores=NC, num_subcores=NS
        ),
        compiler_params=pltpu.CompilerParams(needs_layout_passes=False),
        scratch_types=[
            pltpu.VMEM((L, BPT), jnp.int32),
            pltpu.VMEM((L, BPT), jnp.int32),
            pltpu.VMEM((NBINS, BPT), jnp.float32),
        ],
    )


def _mm_body(ct_ref, t0_ref, t1_ref, out_ref, loss_ref):
    i = pl.program_id(0)
    ct = ct_ref[...]
    zz = jnp.zeros((ZCH_0, DIM), jnp.float32)
    w = jnp.concatenate(
        [
            jnp.concatenate([t0_ref[...], zz], axis=1),
            jnp.concatenate([jnp.zeros((ZCH_1, DIM), jnp.float32), t1_ref[...]], axis=1),
        ],
        axis=0,
    )
    ct_bf = ct.astype(jnp.bfloat16)
    w_hi = w.astype(jnp.bfloat16)
    w_lo = (w - w_hi.astype(jnp.float32)).astype(jnp.bfloat16)
    dn = (((0,), (0,)), ((), ()))
    p = lax.dot_general(ct_bf, w_hi, dn, preferred_element_type=jnp.float32)
    p = p + lax.dot_general(ct_bf, w_lo, dn, preferred_element_type=jnp.float32)
    out_ref[...] = p

    @pl.when(i == 0)
    def _():
        loss_ref[0, 0] = 0.0

    csum = jnp.sum(ct, axis=1, keepdims=True)
    wsum = jnp.sum(w, axis=1, keepdims=True)
    loss_ref[0, 0] += jnp.sum(csum * wsum)


def _pool_matmul(ct, table_0, table_1):
    return pl.pallas_call(
        _mm_body,
        grid=(B // BLK,),
        in_specs=[
            pl.BlockSpec((NBINS, BLK), lambda i: (0, i)),
            pl.BlockSpec((ZCH_0, DIM), lambda i: (0, 0)),
            pl.BlockSpec((ZCH_1, DIM), lambda i: (0, 0)),
        ],
        out_specs=[
            pl.BlockSpec((BLK, 2 * DIM), lambda i: (i, 0)),
            pl.BlockSpec((1, 1), lambda i: (0, 0), memory_space=pltpu.SMEM),
        ],
        out_shape=[
            jax.ShapeDtypeStruct((B, 2 * DIM), jnp.float32),
            jax.ShapeDtypeStruct((1, 1), jnp.float32),
        ],
    )(ct, table_0, table_1)


@jax.jit
def kernel(ids_0, ids_1, table_0, table_1):
    ids0_t = ids_0.astype(jnp.int32).T
    ids1_t = ids_1.astype(jnp.int32).T
    ct = _hist()(ids0_t, ids1_t)
    pred, loss_sum = _pool_matmul(ct, table_0, table_1)
    loss = loss_sum[0, 0] / (B * 2 * DIM)
    return (loss, pred)


# speedup vs baseline: 1.1176x; 1.0001x over previous
import functools

import jax
import jax.numpy as jnp
from jax import lax
from jax.experimental import pallas as pl
from jax.experimental.pallas import tpu as pltpu
from jax.experimental.pallas import tpu_sc as plsc

B, L, DIM = 16384, 20, 128
ZCH_0, ZCH_1 = 8, 16
NBINS = ZCH_0 + ZCH_1

NC, NS, LANES = 2, 16, 16
NW = NC * NS
BPT = B // NW
GROUPS = BPT // LANES

BLK = 8192


def _hist_body(ids0_hbm, ids1_hbm, cnt0_hbm, cnt1_hbm, ids0_v, ids1_v, cnt0_v, cnt1_v):
    wid = lax.axis_index("s") * NC + lax.axis_index("c")
    base = wid * BPT

    pltpu.sync_copy(ids0_hbm.at[:, pl.ds(base, BPT)], ids0_v)
    pltpu.sync_copy(ids1_hbm.at[:, pl.ds(base, BPT)], ids1_v)

    iota = lax.iota(jnp.int32, LANES)
    zeros = jnp.zeros((LANES,), jnp.float32)
    ones = zeros + 1.0

    def zcol(i, _):
        for r in range(ZCH_0):
            cnt0_v[r, pl.ds(i * LANES, LANES)] = zeros
        for r in range(ZCH_1):
            cnt1_v[r, pl.ds(i * LANES, LANES)] = zeros
        return 0

    lax.fori_loop(0, BPT // LANES, zcol, 0)

    def group(g, _):
        lbag = g * LANES + iota
        col = g * LANES
        for l in range(L):
            g0 = ids0_v[l, pl.ds(col, LANES)]
            g1 = ids1_v[l, pl.ds(col, LANES)]
            e0 = g0 & (ZCH_0 - 1)
            e1 = g1 & (ZCH_1 - 1)
            plsc.addupdate_scatter(cnt0_v, [e0, lbag], ones)
            plsc.addupdate_scatter(cnt1_v, [e1, lbag], ones)
        return 0

    lax.fori_loop(0, GROUPS, group, 0)

    pltpu.sync_copy(cnt0_v, cnt0_hbm.at[:, pl.ds(base, BPT)])
    pltpu.sync_copy(cnt1_v, cnt1_hbm.at[:, pl.ds(base, BPT)])


@functools.lru_cache(maxsize=None)
def _hist():
    return pl.kernel(
        _hist_body,
        out_type=[
            jax.ShapeDtypeStruct((ZCH_0, B), jnp.float32),
            jax.ShapeDtypeStruct((ZCH_1, B), jnp.float32),
        ],
        mesh=plsc.VectorSubcoreMesh(
            core_axis_name="c", subcore_axis_name="s", num_cores=NC, num_subcores=NS
        ),
        compiler_params=pltpu.CompilerParams(needs_layout_passes=False),
        scratch_types=[
            pltpu.VMEM((L, BPT), jnp.int32),
            pltpu.VMEM((L, BPT), jnp.int32),
            pltpu.VMEM((ZCH_0, BPT), jnp.float32),
            pltpu.VMEM((ZCH_1, BPT), jnp.float32),
        ],
    )


def _mm_body(ct0_ref, ct1_ref, t0_ref, t1_ref, out_ref, loss_ref):
    i = pl.program_id(0)
    ct = jnp.concatenate([ct0_ref[...], ct1_ref[...]], axis=0)
    zz = jnp.zeros((ZCH_0, DIM), jnp.float32)
    w = jnp.concatenate(
        [
            jnp.concatenate([t0_ref[...], zz], axis=1),
            jnp.concatenate([jnp.zeros((ZCH_1, DIM), jnp.float32), t1_ref[...]], axis=1),
        ],
        axis=0,
    )
    ct_bf = ct.astype(jnp.bfloat16)
    w_hi = w.astype(jnp.bfloat16)
    w_lo = (w - w_hi.astype(jnp.float32)).astype(jnp.bfloat16)
    dn = (((0,), (0,)), ((), ()))
    p = lax.dot_general(ct_bf, w_hi, dn, preferred_element_type=jnp.float32)
    p = p + lax.dot_general(ct_bf, w_lo, dn, preferred_element_type=jnp.float32)
    out_ref[...] = p

    @pl.when(i == 0)
    def _():
        loss_ref[0, 0] = 0.0

    csum = jnp.sum(ct, axis=1, keepdims=True)
    wsum = jnp.sum(w, axis=1, keepdims=True)
    loss_ref[0, 0] += jnp.sum(csum * wsum)


def _pool_matmul(ct0, ct1, table_0, table_1):
    return pl.pallas_call(
        _mm_body,
        grid=(B // BLK,),
        in_specs=[
            pl.BlockSpec((ZCH_0, BLK), lambda i: (0, i)),
            pl.BlockSpec((ZCH_1, BLK), lambda i: (0, i)),
            pl.BlockSpec((ZCH_0, DIM), lambda i: (0, 0)),
            pl.BlockSpec((ZCH_1, DIM), lambda i: (0, 0)),
        ],
        out_specs=[
            pl.BlockSpec((BLK, 2 * DIM), lambda i: (i, 0)),
            pl.BlockSpec((1, 1), lambda i: (0, 0), memory_space=pltpu.SMEM),
        ],
        out_shape=[
            jax.ShapeDtypeStruct((B, 2 * DIM), jnp.float32),
            jax.ShapeDtypeStruct((1, 1), jnp.float32),
        ],
    )(ct0, ct1, table_0, table_1)


@jax.jit
def kernel(ids_0, ids_1, table_0, table_1):
    ids0_t = ids_0.astype(jnp.int32).T
    ids1_t = ids_1.astype(jnp.int32).T
    ct0, ct1 = _hist()(ids0_t, ids1_t)
    pred, loss_sum = _pool_matmul(ct0, ct1, table_0, table_1)
    loss = loss_sum[0, 0] / (B * 2 * DIM)
    return (loss, pred)
